# baseline (speedup 1.0000x reference)
import functools

import jax
import jax.numpy as jnp
from jax import lax
from jax.experimental import pallas as pl
from jax.experimental.pallas import tpu as pltpu
from jax.experimental.pallas import tpu_sc as plsc

_C = 160


def kernel(x, u):
    n, d = x.shape
    nw = 32
    assert n % _C == 0
    nchunk = n // _C
    kmax = -(-nchunk // nw)

    mesh = plsc.VectorSubcoreMesh(core_axis_name="c", subcore_axis_name="s")

    @functools.partial(
        pl.kernel,
        out_type=jax.ShapeDtypeStruct((n, d), x.dtype),
        mesh=mesh,
        scratch_types=[
            pltpu.VMEM((2, _C, d), jnp.float32),
            pltpu.SemaphoreType.DMA((2,)),
            pltpu.SemaphoreType.DMA((2,)),
        ],
    )
    def _copy(x_hbm, o_hbm, bufs, in_sems, out_sems):
        wid = lax.axis_index("s") * mesh.num_cores + lax.axis_index("c")

        def in_cp(k):
            j = wid + k * nw
            return pltpu.make_async_copy(
                x_hbm.at[pl.ds(j * _C, _C), :], bufs.at[k % 2], in_sems.at[k % 2])

        def out_cp(k):
            j = wid + k * nw
            return pltpu.make_async_copy(
                bufs.at[k % 2], o_hbm.at[pl.ds(j * _C, _C), :], out_sems.at[k % 2])

        def valid(k):
            return wid + k * nw < nchunk

        @pl.when(valid(0))
        def _():
            in_cp(0).start()

        for k in range(kmax):
            if k + 1 < kmax:
                @pl.when(valid(k + 1))
                def _(k=k):
                    if k - 1 >= 0:
                        out_cp(k - 1).wait()
                    in_cp(k + 1).start()

            @pl.when(valid(k))
            def _(k=k):
                in_cp(k).wait()
                out_cp(k).start()

        for j in range(kmax):
            if j + 2 < kmax:
                cond = valid(j) & jnp.logical_not(valid(j + 2))
            else:
                cond = valid(j)

            @pl.when(cond)
            def _(j=j):
                out_cp(j).wait()

    return _copy(x)
